# Initial kernel scaffold; baseline (speedup 1.0000x reference)
import jax
import jax.numpy as jnp
from jax.experimental import pallas as pl


def kernel(x, edge_index, W1_l, W1_r, b1, W2_l, W2_r, b2):
    raise NotImplementedError("write your pallas kernel here")


# speedup vs baseline: 10.8949x; 10.8949x over previous
import functools

import jax
import jax.numpy as jnp
from jax import lax
from jax.experimental import pallas as pl
from jax.experimental.pallas import tpu as pltpu
from jax.experimental.pallas import tpu_sc as plsc

NC = 2
NS = 16
NW = NC * NS
CH = 128

ROW_BLK = 400


def _make_seg_sum(n_nodes, n_acc, nchunk, d):
    rpt = n_acc // NS
    mesh = plsc.VectorSubcoreMesh(core_axis_name="c", subcore_axis_name="s")

    @functools.partial(
        pl.kernel,
        mesh=mesh,
        compiler_params=pltpu.CompilerParams(use_tc_tiling_on_sc=False),
        out_type=jax.ShapeDtypeStruct((NC, n_acc, d), jnp.float32),
        scratch_types=[
            pltpu.VMEM((nchunk, CH), jnp.int32),
            pltpu.VMEM((nchunk, CH), jnp.int32),
            pltpu.VMEM((CH, d), jnp.float32),
            pltpu.VMEM((rpt, d), jnp.float32),
            pltpu.VMEM_SHARED((n_acc, d), jnp.float32),
            pltpu.SemaphoreType.DMA,
        ],
    )
    def seg_sum(src_hbm, dst_hbm, p_hbm, zeros_hbm, out_hbm,
                src_v, dst_v, rows_v, obuf_v, acc_sh, sem):
        cid = lax.axis_index("c")
        sid = lax.axis_index("s")
        wid = sid * NC + cid
        r0 = sid * rpt
        pltpu.sync_copy(zeros_hbm, obuf_v)
        pltpu.sync_copy(obuf_v, acc_sh.at[pl.ds(r0, rpt)])
        pltpu.sync_copy(src_hbm.at[wid], src_v)
        pltpu.sync_copy(dst_hbm.at[wid], dst_v)
        plsc.subcore_barrier()

        def body(j, carry):
            pltpu.async_copy(p_hbm.at[src_v.at[j]], rows_v, sem).wait()
            pltpu.sync_copy(rows_v, acc_sh.at[dst_v.at[j]], add=True)
            return carry

        lax.fori_loop(0, nchunk, body, 0)
        plsc.subcore_barrier()
        pltpu.sync_copy(acc_sh.at[pl.ds(r0, rpt)], obuf_v)
        pltpu.sync_copy(obuf_v, out_hbm.at[cid, pl.ds(r0, rpt)])

    return seg_sum


def _proj1_body(x_ref, wl_ref, wr_ref, p_ref, s_ref):
    xb = x_ref[...]
    y1 = jnp.dot(xb, wl_ref[...], preferred_element_type=jnp.float32)
    ones_col = (lax.broadcasted_iota(jnp.int32, y1.shape, 1) == 20).astype(
        jnp.float32)
    p_ref[...] = y1 + ones_col
    s_ref[...] = jnp.dot(xb, wr_ref[...], preferred_element_type=jnp.float32)


def _mid_body(part_ref, s1_ref, b1_ref, w2l_ref, w2r_ref,
              p2_ref, s2_ref, di_ref):
    a = part_ref[0] + part_ref[1]
    di = 1.0 / jnp.maximum(a[:, 20:21], 1.0)
    h = jnp.maximum(a[:, :20] * di + b1_ref[...] + s1_ref[...], 0.0)
    p2_ref[...] = jnp.dot(h, w2l_ref[...], preferred_element_type=jnp.float32)
    s2_ref[...] = jnp.dot(h, w2r_ref[...], preferred_element_type=jnp.float32)
    di_ref[...] = di


def _fin_body(part_ref, s2_ref, di_ref, b2_ref, o_ref):
    a = part_ref[0] + part_ref[1]
    o_ref[...] = a[:, :3] * di_ref[...] + b2_ref[...] + s2_ref[...]


def kernel(x, edge_index, W1_l, W1_r, b1, W2_l, W2_r, b2):
    n, f_in = x.shape
    e = edge_index.shape[1]
    d1 = W1_l.shape[1]
    d2 = W2_l.shape[1]
    dp1, dp2 = 32, 16

    e_pad = -(-e // (NW * CH)) * (NW * CH)
    nchunk = e_pad // (NW * CH)
    n_acc = -(-(n + 1) // (NS * 8)) * (NS * 8)
    rpt = n_acc // NS
    grid = n // ROW_BLK

    ei = edge_index.astype(jnp.int32)
    src = jnp.concatenate([ei[0], jnp.zeros((e_pad - e,), jnp.int32)])
    dst = jnp.concatenate([ei[1], jnp.full((e_pad - e,), n, jnp.int32)])
    src = src.reshape(NW, nchunk, CH)
    dst = dst.reshape(NW, nchunk, CH)
    zeros1 = jnp.zeros((rpt, dp1), jnp.float32)
    zeros2 = jnp.zeros((rpt, dp2), jnp.float32)
    wl_pad = jnp.pad(W1_l, ((0, 0), (0, dp1 - d1)))
    w2l_pad = jnp.pad(W2_l, ((0, 0), (0, dp2 - d2)))

    p1, s1 = pl.pallas_call(
        _proj1_body,
        grid=(grid,),
        in_specs=[
            pl.BlockSpec((ROW_BLK, f_in), lambda i: (i, 0)),
            pl.BlockSpec((f_in, dp1), lambda i: (0, 0)),
            pl.BlockSpec((f_in, d1), lambda i: (0, 0)),
        ],
        out_specs=[
            pl.BlockSpec((ROW_BLK, dp1), lambda i: (i, 0)),
            pl.BlockSpec((ROW_BLK, d1), lambda i: (i, 0)),
        ],
        out_shape=[
            jax.ShapeDtypeStruct((n, dp1), jnp.float32),
            jax.ShapeDtypeStruct((n, d1), jnp.float32),
        ],
    )(x, wl_pad, W1_r)

    part1 = _make_seg_sum(n, n_acc, nchunk, dp1)(src, dst, p1, zeros1)

    p2, s2, di = pl.pallas_call(
        _mid_body,
        grid=(grid,),
        in_specs=[
            pl.BlockSpec((NC, ROW_BLK, dp1), lambda i: (0, i, 0)),
            pl.BlockSpec((ROW_BLK, d1), lambda i: (i, 0)),
            pl.BlockSpec((1, d1), lambda i: (0, 0)),
            pl.BlockSpec((d1, dp2), lambda i: (0, 0)),
            pl.BlockSpec((d1, d2), lambda i: (0, 0)),
        ],
        out_specs=[
            pl.BlockSpec((ROW_BLK, dp2), lambda i: (i, 0)),
            pl.BlockSpec((ROW_BLK, d2), lambda i: (i, 0)),
            pl.BlockSpec((ROW_BLK, 1), lambda i: (i, 0)),
        ],
        out_shape=[
            jax.ShapeDtypeStruct((n, dp2), jnp.float32),
            jax.ShapeDtypeStruct((n, d2), jnp.float32),
            jax.ShapeDtypeStruct((n, 1), jnp.float32),
        ],
    )(part1, s1, b1.reshape(1, d1), w2l_pad, W2_r)

    part2 = _make_seg_sum(n, n_acc, nchunk, dp2)(src, dst, p2, zeros2)

    out = pl.pallas_call(
        _fin_body,
        grid=(grid,),
        in_specs=[
            pl.BlockSpec((NC, ROW_BLK, dp2), lambda i: (0, i, 0)),
            pl.BlockSpec((ROW_BLK, d2), lambda i: (i, 0)),
            pl.BlockSpec((ROW_BLK, 1), lambda i: (i, 0)),
            pl.BlockSpec((1, d2), lambda i: (0, 0)),
        ],
        out_specs=pl.BlockSpec((ROW_BLK, d2), lambda i: (i, 0)),
        out_shape=jax.ShapeDtypeStruct((n, d2), jnp.float32),
    )(part2, s2, di, b2.reshape(1, d2))

    return out
